# baseline (speedup 1.0000x reference)
import functools

import jax
import jax.numpy as jnp
from jax import lax
from jax.experimental import pallas as pl
from jax.experimental.pallas import tpu as pltpu
from jax.experimental.pallas import tpu_sc as plsc

N = 10000
E = 320000
D_IN = 256
H1 = 128
H2 = 64

NC = 2
NS = 16
NW = NC * NS
EC = 64
ROWS_PER_W = 160
E_ROWS_PAD = ROWS_PER_W * NW
E_PAD = E_ROWS_PAD * EC
NBUF = 4
N_PAD = 10240
N_SUB = N_PAD // NS
GRP = 16


R0 = 256
R1 = 64
PAIR_SHIFT = 14
PAIR_BASE = 1 << PAIR_SHIFT
G0 = R0 // GRP
G1 = R1 // GRP


def _seg_sum_body(table, pair2d, zrows, agg_out,
                  pair_v, src_v, dst_v, rows_v, agg_sh,
                  sem_a, sem_b, sem_c, sem_d):
    c = lax.axis_index("c")
    s = lax.axis_index("s")

    for t in range(N_SUB // EC):
        pltpu.sync_copy(zrows, agg_sh.at[pl.ds(s * N_SUB + t * EC, EC)])
    plsc.subcore_barrier()

    my_groups = jnp.where(c == 0, G0, G1)
    base = jnp.where(c == 0, s * R0, NS * R0 + s * R1)

    def group(g, carry):
        @pl.when(g < my_groups)
        def _():
            pltpu.sync_copy(pair2d.at[pl.ds(base + g * GRP, GRP)], pair_v)

            def unpack(t, c2):
                j = t // (EC // 16)
                k = (t % (EC // 16)) * 16
                v = pair_v[j, pl.ds(k, 16)]
                src_v[j, pl.ds(k, 16)] = jnp.bitwise_and(v, PAIR_BASE - 1)
                dst_v[j, pl.ds(k, 16)] = lax.shift_right_logical(v,
                                                                 PAIR_SHIFT)
                return c2

            lax.fori_loop(0, GRP * (EC // 16), unpack, 0)

            sems = (sem_a, sem_b, sem_c, sem_d)
            pend = [None] * GRP
            for j in range(NBUF - 1):
                pend[j] = pltpu.async_copy(table.at[src_v.at[j]],
                                           rows_v.at[j % NBUF],
                                           sems[j % NBUF])
            for j in range(GRP):
                pend[j].wait()
                if j + NBUF - 1 < GRP:
                    jj = j + NBUF - 1
                    pend[jj] = pltpu.async_copy(table.at[src_v.at[jj]],
                                                rows_v.at[jj % NBUF],
                                                sems[jj % NBUF])
                pltpu.sync_copy(rows_v.at[j % NBUF], agg_sh.at[dst_v.at[j]],
                                add=True)

        return carry

    lax.fori_loop(0, G0, group, 0)
    plsc.subcore_barrier()

    pltpu.sync_copy(agg_sh.at[pl.ds(s * N_SUB, N_SUB)],
                    agg_out.at[c, pl.ds(s * N_SUB, N_SUB)])


def _deg_body(dst2d, ones_hbm, zrows, deg_out,
              dst_v, ones_v, deg_sh):
    c = lax.axis_index("c")
    s = lax.axis_index("s")
    w = s * NC + c
    base = w * (E_ROWS_PAD // NW)

    for t in range(N_SUB // EC):
        pltpu.sync_copy(zrows, deg_sh.at[pl.ds(s * N_SUB + t * EC, EC)])
    pltpu.sync_copy(ones_hbm, ones_v)
    plsc.subcore_barrier()

    def group(g, carry):
        pltpu.sync_copy(dst2d.at[pl.ds(base + g * GRP, GRP)], dst_v)

        def body(j, c2):
            pltpu.sync_copy(ones_v, deg_sh.at[dst_v.at[j]], add=True)
            return c2

        return lax.fori_loop(0, GRP, body, carry)

    lax.fori_loop(0, (E_ROWS_PAD // NW) // GRP, group, 0)
    plsc.subcore_barrier()

    pltpu.sync_copy(deg_sh.at[pl.ds(s * N_SUB, N_SUB)],
                    deg_out.at[c, pl.ds(s * N_SUB, N_SUB)])


@functools.cache
def _sc_kernels(d):
    mesh = plsc.VectorSubcoreMesh(core_axis_name="c", subcore_axis_name="s",
                                  num_cores=NC, num_subcores=NS)
    return pl.kernel(
        _seg_sum_body,
        out_type=jax.ShapeDtypeStruct((NC, N_PAD, d), jnp.float32),
        mesh=mesh,
        scratch_types=[
            pltpu.VMEM((GRP, EC), jnp.int32),
            pltpu.VMEM((GRP, EC), jnp.int32),
            pltpu.VMEM((GRP, EC), jnp.int32),
            pltpu.VMEM((NBUF, EC, d), jnp.float32),
            pltpu.VMEM_SHARED((N_PAD, d), jnp.float32),
            pltpu.SemaphoreType.DMA,
            pltpu.SemaphoreType.DMA,
            pltpu.SemaphoreType.DMA,
            pltpu.SemaphoreType.DMA,
        ],
    )


def _seg_sum(table, pair2d, zrows):
    return _sc_kernels(table.shape[1])(table, pair2d, zrows)


@functools.cache
def _deg_kernel():
    mesh = plsc.VectorSubcoreMesh(core_axis_name="c", subcore_axis_name="s",
                                  num_cores=NC, num_subcores=NS)
    return pl.kernel(
        _deg_body,
        out_type=jax.ShapeDtypeStruct((NC, N_PAD, 16), jnp.float32),
        mesh=mesh,
        scratch_types=[
            pltpu.VMEM((GRP, EC), jnp.int32),
            pltpu.VMEM((EC, 16), jnp.float32),
            pltpu.VMEM_SHARED((N_PAD, 16), jnp.float32),
        ],
    )


_BM = 1000


def _proj_body(x_ref, ws_ref, wn_ref, b_ref, xs_ref, xn_ref):
    xv = x_ref[...]
    xs_ref[...] = jnp.dot(xv, ws_ref[...],
                          preferred_element_type=jnp.float32) + b_ref[...]
    xn_ref[...] = jnp.dot(xv, wn_ref[...], preferred_element_type=jnp.float32)


def _tc_proj(x, w_self, w_neigh, b):
    return pl.pallas_call(
        _proj_body,
        grid=(N // _BM,),
        in_specs=[
            pl.BlockSpec((_BM, D_IN), lambda i: (i, 0)),
            pl.BlockSpec((D_IN, H1), lambda i: (0, 0)),
            pl.BlockSpec((D_IN, H1), lambda i: (0, 0)),
            pl.BlockSpec((1, H1), lambda i: (0, 0)),
        ],
        out_specs=[
            pl.BlockSpec((_BM, H1), lambda i: (i, 0)),
            pl.BlockSpec((_BM, H1), lambda i: (i, 0)),
        ],
        out_shape=[
            jax.ShapeDtypeStruct((N, H1), jnp.float32),
            jax.ShapeDtypeStruct((N, H1), jnp.float32),
        ],
    )(x, w_self, w_neigh, b.reshape(1, H1))


def _h_body(xs_ref, agg_ref, deg_ref, w2_ref, b2_ref, w3_ref, b3_ref,
            h_ref, hs2_ref, hs3_ref):
    a = agg_ref[0] + agg_ref[1]
    d = deg_ref[0, :, 0:1] + deg_ref[1, :, 0:1]
    recip = 1.0 / jnp.maximum(d, 1.0)
    h = jnp.maximum(xs_ref[...] + a * recip, 0.0)
    h_ref[...] = h
    hs2_ref[...] = jnp.dot(h, w2_ref[...],
                           preferred_element_type=jnp.float32) + b2_ref[...]
    hs3_ref[...] = jnp.dot(h, w3_ref[...],
                           preferred_element_type=jnp.float32) + b3_ref[...]


def _tc_h(xs, agg, deg, w2s, b2, w3s, b3):
    return pl.pallas_call(
        _h_body,
        grid=(N // _BM,),
        in_specs=[
            pl.BlockSpec((_BM, H1), lambda i: (i, 0)),
            pl.BlockSpec((NC, _BM, H1), lambda i: (0, i, 0)),
            pl.BlockSpec((NC, _BM, 16), lambda i: (0, i, 0)),
            pl.BlockSpec((H1, H2), lambda i: (0, 0)),
            pl.BlockSpec((1, H2), lambda i: (0, 0)),
            pl.BlockSpec((H1, H2), lambda i: (0, 0)),
            pl.BlockSpec((1, H2), lambda i: (0, 0)),
        ],
        out_specs=[
            pl.BlockSpec((_BM, H1), lambda i: (i, 0)),
            pl.BlockSpec((_BM, H2), lambda i: (i, 0)),
            pl.BlockSpec((_BM, H2), lambda i: (i, 0)),
        ],
        out_shape=[
            jax.ShapeDtypeStruct((N, H1), jnp.float32),
            jax.ShapeDtypeStruct((N, H2), jnp.float32),
            jax.ShapeDtypeStruct((N, H2), jnp.float32),
        ],
    )(xs, agg, deg, w2s, b2.reshape(1, H2), w3s, b3.reshape(1, H2))


def _z_body(hs2_ref, hs3_ref, agg_ref, deg_ref, w2n_ref, w3n_ref, eps_ref,
            z_ref):
    a = agg_ref[0] + agg_ref[1]
    d = deg_ref[0, :, 0:1] + deg_ref[1, :, 0:1]
    recip = 1.0 / jnp.maximum(d, 1.0)
    hn = a * recip
    h_mean = hs2_ref[...] + jnp.dot(hn, w2n_ref[...],
                                    preferred_element_type=jnp.float32)
    h_log_std = hs3_ref[...] + jnp.dot(hn, w3n_ref[...],
                                       preferred_element_type=jnp.float32)
    z_ref[...] = h_mean + jnp.exp(h_log_std) * eps_ref[...]


def _tc_z(hs2, hs3, agg, deg, w2n, w3n, eps):
    return pl.pallas_call(
        _z_body,
        grid=(N // _BM,),
        in_specs=[
            pl.BlockSpec((_BM, H2), lambda i: (i, 0)),
            pl.BlockSpec((_BM, H2), lambda i: (i, 0)),
            pl.BlockSpec((NC, _BM, H1), lambda i: (0, i, 0)),
            pl.BlockSpec((NC, _BM, 16), lambda i: (0, i, 0)),
            pl.BlockSpec((H1, H2), lambda i: (0, 0)),
            pl.BlockSpec((H1, H2), lambda i: (0, 0)),
            pl.BlockSpec((_BM, H2), lambda i: (i, 0)),
        ],
        out_specs=pl.BlockSpec((_BM, H2), lambda i: (i, 0)),
        out_shape=jax.ShapeDtypeStruct((N, H2), jnp.float32),
    )(hs2, hs3, agg, deg, w2n, w3n, eps)


_BDM = 512
_BDN = 2048


def _dec_body(zi_ref, zj_ref, out_ref):
    r = lax.dot_general(zi_ref[...], zj_ref[...],
                        (((1,), (1,)), ((), ())),
                        preferred_element_type=jnp.float32)
    out_ref[...] = jax.nn.sigmoid(r)


def _tc_decoder(z):
    return pl.pallas_call(
        _dec_body,
        grid=(-(-N // _BDM), -(-N // _BDN)),
        in_specs=[
            pl.BlockSpec((_BDM, H2), lambda i, j: (i, 0)),
            pl.BlockSpec((_BDN, H2), lambda i, j: (j, 0)),
        ],
        out_specs=pl.BlockSpec((_BDM, _BDN), lambda i, j: (i, j)),
        out_shape=jax.ShapeDtypeStruct((N, N), jnp.float32),
        compiler_params=pltpu.CompilerParams(
            dimension_semantics=("parallel", "parallel")),
    )(z, z)


def kernel(x, edge_index, W1_self, W1_neigh, b1, W2_self, W2_neigh, b2,
           W3_self, W3_neigh, b3):
    src = edge_index[0].astype(jnp.int32)
    dst = edge_index[1].astype(jnp.int32)
    pad = E_PAD - E
    pair2d = jnp.concatenate(
        [dst * PAIR_BASE + src,
         jnp.full((pad,), N * PAIR_BASE, jnp.int32)]).reshape(E_ROWS_PAD, EC)
    dst2d = jnp.concatenate(
        [dst, jnp.full((pad,), N, jnp.int32)]).reshape(E_ROWS_PAD, EC)
    zrows = jnp.zeros((EC, H1), jnp.float32)
    ones = jnp.ones((EC, 16), jnp.float32)
    zdeg = jnp.zeros((EC, 16), jnp.float32)

    xs, xn = _tc_proj(x, W1_self, W1_neigh, b1)
    deg = _deg_kernel()(dst2d, ones, zdeg)
    agg1 = _seg_sum(xn, pair2d, zrows)
    h, hs2, hs3 = _tc_h(xs, agg1, deg, W2_self, b2, W3_self, b3)
    agg2 = _seg_sum(h, pair2d, zrows)
    eps = jax.random.normal(jax.random.key(42), (N, H2), dtype=jnp.float32)
    z = _tc_z(hs2, hs3, agg2, deg, W2_neigh, W3_neigh, eps)
    return _tc_decoder(z)


# speedup vs baseline: 1.0698x; 1.0698x over previous
import functools

import jax
import jax.numpy as jnp
from jax import lax
from jax.experimental import pallas as pl
from jax.experimental.pallas import tpu as pltpu
from jax.experimental.pallas import tpu_sc as plsc

N = 10000
E = 320000
D_IN = 256
H1 = 128
H2 = 64

NC = 2
NS = 16
NW = NC * NS
EC = 64
ROWS_PER_W = 160
E_ROWS_PAD = ROWS_PER_W * NW
E_PAD = E_ROWS_PAD * EC
NBUF = 4
N_PAD = 10240
N_SUB = N_PAD // NS
GRP = 16


R0 = 216
R1 = 104
PAIR_SHIFT = 14
PAIR_BASE = 1 << PAIR_SHIFT
G0 = R0 // GRP
G1 = R1 // GRP


def _seg_sum_body(table, pair2d, zrows, agg_out,
                  pair_v, src_v, dst_v, rows_v, agg_sh,
                  sem_a, sem_b, sem_c, sem_d):
    c = lax.axis_index("c")
    s = lax.axis_index("s")

    for t in range(N_SUB // EC):
        pltpu.sync_copy(zrows, agg_sh.at[pl.ds(s * N_SUB + t * EC, EC)])
    plsc.subcore_barrier()

    my_groups = jnp.where(c == 0, G0, G1)
    base = jnp.where(c == 0, s * R0, NS * R0 + s * R1)

    def group(g, carry):
        @pl.when(g < my_groups)
        def _():
            pltpu.sync_copy(pair2d.at[pl.ds(base + g * GRP, GRP)], pair_v)

            def unpack(t, c2):
                j = t // (EC // 16)
                k = (t % (EC // 16)) * 16
                v = pair_v[j, pl.ds(k, 16)]
                src_v[j, pl.ds(k, 16)] = jnp.bitwise_and(v, PAIR_BASE - 1)
                dst_v[j, pl.ds(k, 16)] = lax.shift_right_logical(v,
                                                                 PAIR_SHIFT)
                return c2

            lax.fori_loop(0, GRP * (EC // 16), unpack, 0)

            sems = (sem_a, sem_b, sem_c, sem_d)
            pend = [None] * GRP
            for j in range(NBUF - 1):
                pend[j] = pltpu.async_copy(table.at[src_v.at[j]],
                                           rows_v.at[j % NBUF],
                                           sems[j % NBUF])
            for j in range(GRP):
                pend[j].wait()
                if j + NBUF - 1 < GRP:
                    jj = j + NBUF - 1
                    pend[jj] = pltpu.async_copy(table.at[src_v.at[jj]],
                                                rows_v.at[jj % NBUF],
                                                sems[jj % NBUF])
                pltpu.sync_copy(rows_v.at[j % NBUF], agg_sh.at[dst_v.at[j]],
                                add=True)

        return carry

    lax.fori_loop(0, G0, group, 0)
    plsc.subcore_barrier()

    pltpu.sync_copy(agg_sh.at[pl.ds(s * N_SUB, N_SUB)],
                    agg_out.at[c, pl.ds(s * N_SUB, N_SUB)])


def _deg_body(dst2d, ones_hbm, zrows, deg_out,
              dst_v, ones_v, deg_sh):
    c = lax.axis_index("c")
    s = lax.axis_index("s")
    w = s * NC + c
    base = w * (E_ROWS_PAD // NW)

    for t in range(N_SUB // EC):
        pltpu.sync_copy(zrows, deg_sh.at[pl.ds(s * N_SUB + t * EC, EC)])
    pltpu.sync_copy(ones_hbm, ones_v)
    plsc.subcore_barrier()

    def group(g, carry):
        pltpu.sync_copy(dst2d.at[pl.ds(base + g * GRP, GRP)], dst_v)

        def body(j, c2):
            pltpu.sync_copy(ones_v, deg_sh.at[dst_v.at[j]], add=True)
            return c2

        return lax.fori_loop(0, GRP, body, carry)

    lax.fori_loop(0, (E_ROWS_PAD // NW) // GRP, group, 0)
    plsc.subcore_barrier()

    pltpu.sync_copy(deg_sh.at[pl.ds(s * N_SUB, N_SUB)],
                    deg_out.at[c, pl.ds(s * N_SUB, N_SUB)])


@functools.cache
def _sc_kernels(d):
    mesh = plsc.VectorSubcoreMesh(core_axis_name="c", subcore_axis_name="s",
                                  num_cores=NC, num_subcores=NS)
    return pl.kernel(
        _seg_sum_body,
        out_type=jax.ShapeDtypeStruct((NC, N_PAD, d), jnp.float32),
        mesh=mesh,
        scratch_types=[
            pltpu.VMEM((GRP, EC), jnp.int32),
            pltpu.VMEM((GRP, EC), jnp.int32),
            pltpu.VMEM((GRP, EC), jnp.int32),
            pltpu.VMEM((NBUF, EC, d), jnp.float32),
            pltpu.VMEM_SHARED((N_PAD, d), jnp.float32),
            pltpu.SemaphoreType.DMA,
            pltpu.SemaphoreType.DMA,
            pltpu.SemaphoreType.DMA,
            pltpu.SemaphoreType.DMA,
        ],
    )


def _seg_sum(table, pair2d, zrows):
    return _sc_kernels(table.shape[1])(table, pair2d, zrows)


@functools.cache
def _deg_kernel():
    mesh = plsc.VectorSubcoreMesh(core_axis_name="c", subcore_axis_name="s",
                                  num_cores=NC, num_subcores=NS)
    return pl.kernel(
        _deg_body,
        out_type=jax.ShapeDtypeStruct((NC, N_PAD, 16), jnp.float32),
        mesh=mesh,
        scratch_types=[
            pltpu.VMEM((GRP, EC), jnp.int32),
            pltpu.VMEM((EC, 16), jnp.float32),
            pltpu.VMEM_SHARED((N_PAD, 16), jnp.float32),
        ],
    )


_BM = 1000


def _proj_body(x_ref, ws_ref, wn_ref, b_ref, xs_ref, xn_ref):
    xv = x_ref[...]
    xs_ref[...] = jnp.dot(xv, ws_ref[...],
                          preferred_element_type=jnp.float32) + b_ref[...]
    xn_ref[...] = jnp.dot(xv, wn_ref[...], preferred_element_type=jnp.float32)


def _tc_proj(x, w_self, w_neigh, b):
    return pl.pallas_call(
        _proj_body,
        grid=(N // _BM,),
        in_specs=[
            pl.BlockSpec((_BM, D_IN), lambda i: (i, 0)),
            pl.BlockSpec((D_IN, H1), lambda i: (0, 0)),
            pl.BlockSpec((D_IN, H1), lambda i: (0, 0)),
            pl.BlockSpec((1, H1), lambda i: (0, 0)),
        ],
        out_specs=[
            pl.BlockSpec((_BM, H1), lambda i: (i, 0)),
            pl.BlockSpec((_BM, H1), lambda i: (i, 0)),
        ],
        out_shape=[
            jax.ShapeDtypeStruct((N, H1), jnp.float32),
            jax.ShapeDtypeStruct((N, H1), jnp.float32),
        ],
    )(x, w_self, w_neigh, b.reshape(1, H1))


def _h_body(xs_ref, agg_ref, deg_ref, w2_ref, b2_ref, w3_ref, b3_ref,
            h_ref, hs2_ref, hs3_ref):
    a = agg_ref[0] + agg_ref[1]
    d = deg_ref[0, :, 0:1] + deg_ref[1, :, 0:1]
    recip = 1.0 / jnp.maximum(d, 1.0)
    h = jnp.maximum(xs_ref[...] + a * recip, 0.0)
    h_ref[...] = h
    hs2_ref[...] = jnp.dot(h, w2_ref[...],
                           preferred_element_type=jnp.float32) + b2_ref[...]
    hs3_ref[...] = jnp.dot(h, w3_ref[...],
                           preferred_element_type=jnp.float32) + b3_ref[...]


def _tc_h(xs, agg, deg, w2s, b2, w3s, b3):
    return pl.pallas_call(
        _h_body,
        grid=(N // _BM,),
        in_specs=[
            pl.BlockSpec((_BM, H1), lambda i: (i, 0)),
            pl.BlockSpec((NC, _BM, H1), lambda i: (0, i, 0)),
            pl.BlockSpec((NC, _BM, 16), lambda i: (0, i, 0)),
            pl.BlockSpec((H1, H2), lambda i: (0, 0)),
            pl.BlockSpec((1, H2), lambda i: (0, 0)),
            pl.BlockSpec((H1, H2), lambda i: (0, 0)),
            pl.BlockSpec((1, H2), lambda i: (0, 0)),
        ],
        out_specs=[
            pl.BlockSpec((_BM, H1), lambda i: (i, 0)),
            pl.BlockSpec((_BM, H2), lambda i: (i, 0)),
            pl.BlockSpec((_BM, H2), lambda i: (i, 0)),
        ],
        out_shape=[
            jax.ShapeDtypeStruct((N, H1), jnp.float32),
            jax.ShapeDtypeStruct((N, H2), jnp.float32),
            jax.ShapeDtypeStruct((N, H2), jnp.float32),
        ],
    )(xs, agg, deg, w2s, b2.reshape(1, H2), w3s, b3.reshape(1, H2))


def _z_body(hs2_ref, hs3_ref, agg_ref, deg_ref, w2n_ref, w3n_ref, eps_ref,
            z_ref):
    a = agg_ref[0] + agg_ref[1]
    d = deg_ref[0, :, 0:1] + deg_ref[1, :, 0:1]
    recip = 1.0 / jnp.maximum(d, 1.0)
    hn = a * recip
    h_mean = hs2_ref[...] + jnp.dot(hn, w2n_ref[...],
                                    preferred_element_type=jnp.float32)
    h_log_std = hs3_ref[...] + jnp.dot(hn, w3n_ref[...],
                                       preferred_element_type=jnp.float32)
    z_ref[...] = h_mean + jnp.exp(h_log_std) * eps_ref[...]


def _tc_z(hs2, hs3, agg, deg, w2n, w3n, eps):
    return pl.pallas_call(
        _z_body,
        grid=(N // _BM,),
        in_specs=[
            pl.BlockSpec((_BM, H2), lambda i: (i, 0)),
            pl.BlockSpec((_BM, H2), lambda i: (i, 0)),
            pl.BlockSpec((NC, _BM, H1), lambda i: (0, i, 0)),
            pl.BlockSpec((NC, _BM, 16), lambda i: (0, i, 0)),
            pl.BlockSpec((H1, H2), lambda i: (0, 0)),
            pl.BlockSpec((H1, H2), lambda i: (0, 0)),
            pl.BlockSpec((_BM, H2), lambda i: (i, 0)),
        ],
        out_specs=pl.BlockSpec((_BM, H2), lambda i: (i, 0)),
        out_shape=jax.ShapeDtypeStruct((N, H2), jnp.float32),
    )(hs2, hs3, agg, deg, w2n, w3n, eps)


_BDM = 512
_BDN = 2048


def _dec_body(zi_ref, zj_ref, out_ref):
    r = lax.dot_general(zi_ref[...], zj_ref[...],
                        (((1,), (1,)), ((), ())),
                        preferred_element_type=jnp.float32)
    out_ref[...] = jax.nn.sigmoid(r)


def _tc_decoder(z):
    return pl.pallas_call(
        _dec_body,
        grid=(-(-N // _BDM), -(-N // _BDN)),
        in_specs=[
            pl.BlockSpec((_BDM, H2), lambda i, j: (i, 0)),
            pl.BlockSpec((_BDN, H2), lambda i, j: (j, 0)),
        ],
        out_specs=pl.BlockSpec((_BDM, _BDN), lambda i, j: (i, j)),
        out_shape=jax.ShapeDtypeStruct((N, N), jnp.float32),
        compiler_params=pltpu.CompilerParams(
            dimension_semantics=("parallel", "parallel")),
    )(z, z)


def kernel(x, edge_index, W1_self, W1_neigh, b1, W2_self, W2_neigh, b2,
           W3_self, W3_neigh, b3):
    src = edge_index[0].astype(jnp.int32)
    dst = edge_index[1].astype(jnp.int32)
    pad = E_PAD - E
    pair2d = jnp.concatenate(
        [dst * PAIR_BASE + src,
         jnp.full((pad,), N * PAIR_BASE, jnp.int32)]).reshape(E_ROWS_PAD, EC)
    dst2d = jnp.concatenate(
        [dst, jnp.full((pad,), N, jnp.int32)]).reshape(E_ROWS_PAD, EC)
    zrows = jnp.zeros((EC, H1), jnp.float32)
    ones = jnp.ones((EC, 16), jnp.float32)
    zdeg = jnp.zeros((EC, 16), jnp.float32)

    xs, xn = _tc_proj(x, W1_self, W1_neigh, b1)
    deg = _deg_kernel()(dst2d, ones, zdeg)
    agg1 = _seg_sum(xn, pair2d, zrows)
    h, hs2, hs3 = _tc_h(xs, agg1, deg, W2_self, b2, W3_self, b3)
    agg2 = _seg_sum(h, pair2d, zrows)
    eps = jax.random.normal(jax.random.key(42), (N, H2), dtype=jnp.float32)
    z = _tc_z(hs2, hs3, agg2, deg, W2_neigh, W3_neigh, eps)
    return _tc_decoder(z)
